# Initial kernel scaffold; baseline (speedup 1.0000x reference)
import functools

import jax
import jax.numpy as jnp
from jax import lax
from jax.experimental import pallas as pl
from jax.experimental.pallas import tpu as pltpu
from jax.experimental.pallas import tpu_sc as plsc

N = 3200000
M = 100000
MP = 100352
NB = N // 128
NW = 32
CH = 64
SL = MP // 16
EM = MP * 3 // NW

_mesh = plsc.VectorSubcoreMesh(core_axis_name="c", subcore_axis_name="s")
_f32 = jnp.float32
_i32 = jnp.int32


def _worker_range(w):
    lo = (w * NB) // NW
    hi = ((w + 1) * NB) // NW
    return lo, hi


@functools.partial(
    pl.kernel,
    out_type=(
        jax.ShapeDtypeStruct((2, MP, 3), _f32),
        jax.ShapeDtypeStruct((2, MP, 3), _f32),
    ),
    mesh=_mesh,
    scratch_types=[
        pltpu.VMEM_SHARED((MP, 3), _f32),
        pltpu.VMEM_SHARED((MP, 3), _f32),
        pltpu.VMEM((CH, 1, 128), _i32),
        pltpu.VMEM((CH, 128, 3), _f32),
        pltpu.VMEM((128, 3), _f32),
    ],
)
def _k_partials(atoms3, ids3, zeros3, ones3, psum_out, pcnt_out,
                ssum, scnt, ids_v, atoms_v, ones_v):
    c = lax.axis_index("c")
    s = lax.axis_index("s")
    w = c * 16 + s

    zsl = pl.ds(s * SL, SL)
    pltpu.sync_copy(zeros3.at[zsl], ssum.at[zsl])
    pltpu.sync_copy(zeros3.at[zsl], scnt.at[zsl])
    pltpu.sync_copy(ones3, ones_v)
    plsc.subcore_barrier()

    lo, hi = _worker_range(w)
    nf = (hi - lo) // CH

    def chunk(k, _):
        base = lo + k * CH
        pltpu.sync_copy(ids3.at[pl.ds(base, CH)], ids_v)
        pltpu.sync_copy(atoms3.at[pl.ds(base, CH)], atoms_v)

        def blk(j, _):
            idx = ids_v.at[j, 0]
            pltpu.sync_copy(atoms_v.at[j], ssum.at[idx], add=True)
            pltpu.sync_copy(ones_v, scnt.at[idx], add=True)
            return 0

        lax.fori_loop(0, CH, blk, 0)
        return 0

    lax.fori_loop(0, nf, chunk, 0)

    tl = lo + nf * CH

    def tblk(j, _):
        pltpu.sync_copy(ids3.at[pl.ds(tl + j, 1)], ids_v.at[pl.ds(0, 1)])
        pltpu.sync_copy(atoms3.at[pl.ds(tl + j, 1)], atoms_v.at[pl.ds(0, 1)])
        idx = ids_v.at[0, 0]
        pltpu.sync_copy(atoms_v.at[0], ssum.at[idx], add=True)
        pltpu.sync_copy(ones_v, scnt.at[idx], add=True)
        return 0

    lax.fori_loop(0, hi - tl, tblk, 0)

    plsc.subcore_barrier()
    pltpu.sync_copy(ssum.at[zsl], psum_out.at[c, zsl])
    pltpu.sync_copy(scnt.at[zsl], pcnt_out.at[c, zsl])


@functools.partial(
    pl.kernel,
    out_type=jax.ShapeDtypeStruct((MP * 3,), _f32),
    mesh=_mesh,
    scratch_types=[
        pltpu.VMEM((EM,), _f32),
        pltpu.VMEM((EM,), _f32),
        pltpu.VMEM((EM,), _f32),
        pltpu.VMEM((EM,), _f32),
    ],
)
def _k_means(ps_f, pc_f, mout, s0, s1, c0, c1):
    c = lax.axis_index("c")
    s = lax.axis_index("s")
    w = c * 16 + s
    off = pl.ds(w * EM, EM)
    pltpu.sync_copy(ps_f.at[0, off], s0)
    pltpu.sync_copy(ps_f.at[1, off], s1)
    pltpu.sync_copy(pc_f.at[0, off], c0)
    pltpu.sync_copy(pc_f.at[1, off], c1)

    def vec(t, _):
        sl = pl.ds(t * 16, 16)
        sm = s0[sl] + s1[sl]
        ct = c0[sl] + c1[sl]
        s0[sl] = sm / jnp.maximum(ct, 1.0)
        return 0

    lax.fori_loop(0, EM // 16, vec, 0)
    pltpu.sync_copy(s0, mout.at[off])


@functools.partial(
    pl.kernel,
    out_type=jax.ShapeDtypeStruct((N * 3,), _f32),
    mesh=_mesh,
    scratch_types=[
        pltpu.VMEM((CH, 1, 128), _i32),
        pltpu.VMEM((CH * 384,), _f32),
        pltpu.VMEM((CH * 384,), _f32),
        pltpu.VMEM((128, 3), _f32),
        pltpu.VMEM((384,), _i32),
        pltpu.VMEM((384,), _i32),
        pltpu.SemaphoreType.DMA,
    ],
)
def _k_center(atoms_f, ids3, means3, ivt_h, cvt_h, out_f,
              ids_v, atoms_v, out_v, mrows, ivt, cvt, sem):
    c = lax.axis_index("c")
    s = lax.axis_index("s")
    w = c * 16 + s
    pltpu.sync_copy(ivt_h, ivt)
    pltpu.sync_copy(cvt_h, cvt)

    lo, hi = _worker_range(w)
    nf = (hi - lo) // CH

    def centered_block(j, _):
        pltpu.async_copy(means3.at[ids_v.at[j, 0]], mrows, sem).wait()
        for t in range(24):
            sl = pl.ds(j * 384 + t * 16, 16)
            iv = ivt[pl.ds(t * 16, 16)]
            cv = cvt[pl.ds(t * 16, 16)]
            m = plsc.load_gather(mrows, [iv, cv])
            out_v[sl] = atoms_v[sl] - m
        return 0

    def chunk(k, _):
        base = lo + k * CH
        fsl = pl.ds(base * 384, CH * 384)
        pltpu.sync_copy(ids3.at[pl.ds(base, CH)], ids_v)
        pltpu.sync_copy(atoms_f.at[fsl], atoms_v)
        lax.fori_loop(0, CH, centered_block, 0)
        pltpu.sync_copy(out_v, out_f.at[fsl])
        return 0

    lax.fori_loop(0, nf, chunk, 0)

    tl = lo + nf * CH

    def tblk(j, _):
        bsl = pl.ds((tl + j) * 384, 384)
        pltpu.sync_copy(ids3.at[pl.ds(tl + j, 1)], ids_v.at[pl.ds(0, 1)])
        pltpu.sync_copy(atoms_f.at[bsl], atoms_v.at[pl.ds(0, 384)])
        lax.fori_loop(0, 1, centered_block, 0)
        pltpu.sync_copy(out_v.at[pl.ds(0, 384)], out_f.at[bsl])
        return 0

    lax.fori_loop(0, hi - tl, tblk, 0)


def kernel(atoms_x, graph_batch):
    atoms3 = atoms_x.reshape(NB, 128, 3)
    atoms_f = atoms_x.reshape(N * 3)
    ids3 = graph_batch.reshape(NB, 1, 128)
    zeros3 = jnp.zeros((MP, 3), _f32)
    ones3 = jnp.ones((128, 3), _f32)
    e = jnp.arange(384, dtype=_i32)
    ivt = e // 3
    cvt = e - 3 * ivt

    psum, pcnt = _k_partials(atoms3, ids3, zeros3, ones3)
    means_f = _k_means(psum.reshape(2, MP * 3), pcnt.reshape(2, MP * 3))
    out_f = _k_center(atoms_f, ids3, means_f.reshape(MP, 3), ivt, cvt)
    return out_f.reshape(N, 3)


# speedup vs baseline: 2.0489x; 2.0489x over previous
import functools

import jax
import jax.numpy as jnp
from jax import lax
from jax.experimental import pallas as pl
from jax.experimental.pallas import tpu as pltpu
from jax.experimental.pallas import tpu_sc as plsc

N = 3200000
M = 100000
MP = 100352
NB = N // 128
NW = 32
CH = 64
SL = MP // 16
EM = MP * 8 // NW

_mesh = plsc.VectorSubcoreMesh(core_axis_name="c", subcore_axis_name="s")
_params = pltpu.CompilerParams(use_tc_tiling_on_sc=False, needs_layout_passes=False)
_f32 = jnp.float32
_i32 = jnp.int32


def _worker_range(w):
    lo = (w * NB) // NW
    hi = ((w + 1) * NB) // NW
    return lo, hi


@functools.partial(
    pl.kernel,
    out_type=jax.ShapeDtypeStruct((2, MP, 8), _f32),
    mesh=_mesh,
    compiler_params=_params,
    scratch_types=[
        pltpu.VMEM_SHARED((MP, 8), _f32),
        pltpu.VMEM((CH, 1, 128), _i32),
        pltpu.VMEM((CH * 384,), _f32),
        pltpu.VMEM((128, 8), _f32),
        pltpu.VMEM((16,), _i32),
        pltpu.VMEM((16,), _i32),
        pltpu.VMEM((16,), _i32),
        pltpu.VMEM((16,), _f32),
        pltpu.VMEM((16,), _f32),
    ],
)
def _k_partials(atoms_f, ids3, zeros8, pg_h, piv_h, pcv_h, mc_h, m1_h, pout,
                acc, ids_v, atoms_v, vals_v, pg_v, piv_v, pcv_v, mc_v, m1_v):
    c = lax.axis_index("c")
    s = lax.axis_index("s")
    w = c * 16 + s

    zsl = pl.ds(s * SL, SL)
    pltpu.sync_copy(zeros8.at[zsl], acc.at[zsl])
    pltpu.sync_copy(pg_h, pg_v)
    pltpu.sync_copy(piv_h, piv_v)
    pltpu.sync_copy(pcv_h, pcv_v)
    pltpu.sync_copy(mc_h, mc_v)
    pltpu.sync_copy(m1_h, m1_v)
    plsc.subcore_barrier()

    pg = pg_v[...]
    piv = piv_v[...]
    pcv = pcv_v[...]
    mc = mc_v[...]
    m1 = m1_v[...]

    lo, hi = _worker_range(w)
    nf = (hi - lo) // CH

    def do_block(j):
        abase = j * 384
        for t in range(64):
            g = plsc.load_gather(atoms_v, [pg + (abase + 6 * t)])
            v = g * mc + m1
            plsc.store_scatter(vals_v, [piv + 2 * t, pcv], v)
        pltpu.sync_copy(vals_v, acc.at[ids_v.at[j, 0]], add=True)

    def chunk(k, _):
        base = lo + k * CH
        pltpu.sync_copy(ids3.at[pl.ds(base, CH)], ids_v)
        pltpu.sync_copy(atoms_f.at[pl.ds(base * 384, CH * 384)], atoms_v)

        def blk(j, _):
            do_block(j)
            return 0

        lax.fori_loop(0, CH, blk, 0)
        return 0

    lax.fori_loop(0, nf, chunk, 0)

    tl = lo + nf * CH

    def tblk(j, _):
        pltpu.sync_copy(ids3.at[pl.ds(tl + j, 1)], ids_v.at[pl.ds(0, 1)])
        pltpu.sync_copy(atoms_f.at[pl.ds((tl + j) * 384, 384)],
                        atoms_v.at[pl.ds(0, 384)])
        do_block(0)
        return 0

    lax.fori_loop(0, hi - tl, tblk, 0)

    plsc.subcore_barrier()
    pltpu.sync_copy(acc.at[zsl], pout.at[c, zsl])


@functools.partial(
    pl.kernel,
    out_type=jax.ShapeDtypeStruct((MP * 8,), _f32),
    mesh=_mesh,
    compiler_params=_params,
    scratch_types=[
        pltpu.VMEM((EM,), _f32),
        pltpu.VMEM((EM,), _f32),
        pltpu.VMEM((16,), _i32),
    ],
)
def _k_means(p_f, pc_h, mout, s0, s1, pc_v):
    c = lax.axis_index("c")
    s = lax.axis_index("s")
    w = c * 16 + s
    off = pl.ds(w * EM, EM)
    pltpu.sync_copy(p_f.at[0, off], s0)
    pltpu.sync_copy(p_f.at[1, off], s1)
    pltpu.sync_copy(pc_h, pc_v)
    pc = pc_v[...]

    def add2(t, _):
        sl = pl.ds(t * 16, 16)
        s0[sl] = s0[sl] + s1[sl]
        return 0

    lax.fori_loop(0, EM // 16, add2, 0)

    def mean(t, _):
        sl = pl.ds(t * 16, 16)
        cnt = plsc.load_gather(s0, [pc + 16 * t])
        s0[sl] = s0[sl] / jnp.maximum(cnt, 1.0)
        return 0

    lax.fori_loop(0, EM // 16, mean, 0)
    pltpu.sync_copy(s0, mout.at[off])


@functools.partial(
    pl.kernel,
    out_type=jax.ShapeDtypeStruct((N * 3,), _f32),
    mesh=_mesh,
    compiler_params=_params,
    scratch_types=[
        pltpu.VMEM((CH, 1, 128), _i32),
        pltpu.VMEM((CH * 384,), _f32),
        pltpu.VMEM((CH * 384,), _f32),
        pltpu.VMEM((128, 8), _f32),
        pltpu.VMEM((384,), _i32),
        pltpu.VMEM((384,), _i32),
        pltpu.SemaphoreType.DMA,
    ],
)
def _k_center(atoms_f, ids3, means8, ivt_h, cvt_h, out_f,
              ids_v, atoms_v, out_v, mrows, ivt, cvt, sem):
    c = lax.axis_index("c")
    s = lax.axis_index("s")
    w = c * 16 + s
    pltpu.sync_copy(ivt_h, ivt)
    pltpu.sync_copy(cvt_h, cvt)

    lo, hi = _worker_range(w)
    nf = (hi - lo) // CH

    def centered_block(j, _):
        pltpu.async_copy(means8.at[ids_v.at[j, 0]], mrows, sem).wait()
        for t in range(24):
            sl = pl.ds(j * 384 + t * 16, 16)
            iv = ivt[pl.ds(t * 16, 16)]
            cv = cvt[pl.ds(t * 16, 16)]
            m = plsc.load_gather(mrows, [iv, cv])
            out_v[sl] = atoms_v[sl] - m
        return 0

    def chunk(k, _):
        base = lo + k * CH
        fsl = pl.ds(base * 384, CH * 384)
        pltpu.sync_copy(ids3.at[pl.ds(base, CH)], ids_v)
        pltpu.sync_copy(atoms_f.at[fsl], atoms_v)
        lax.fori_loop(0, CH, centered_block, 0)
        pltpu.sync_copy(out_v, out_f.at[fsl])
        return 0

    lax.fori_loop(0, nf, chunk, 0)

    tl = lo + nf * CH

    def tblk(j, _):
        bsl = pl.ds((tl + j) * 384, 384)
        pltpu.sync_copy(ids3.at[pl.ds(tl + j, 1)], ids_v.at[pl.ds(0, 1)])
        pltpu.sync_copy(atoms_f.at[bsl], atoms_v.at[pl.ds(0, 384)])
        lax.fori_loop(0, 1, centered_block, 0)
        pltpu.sync_copy(out_v.at[pl.ds(0, 384)], out_f.at[bsl])
        return 0

    lax.fori_loop(0, hi - tl, tblk, 0)


def kernel(atoms_x, graph_batch):
    atoms_f = atoms_x.reshape(N * 3)
    ids3 = graph_batch.reshape(NB, 1, 128)
    zeros8 = jnp.zeros((MP, 8), _f32)

    lane = jnp.arange(16, dtype=_i32)
    cc = lane % 8
    pg = 3 * (lane // 8) + jnp.minimum(cc, 2)
    piv = lane // 8
    pcv = cc
    mc = jnp.where(cc < 3, 1.0, 0.0).astype(_f32)
    m1 = jnp.where(cc == 3, 1.0, 0.0).astype(_f32)

    pc = 8 * (lane // 8) + 3

    e = jnp.arange(384, dtype=_i32)
    ivt = e // 3
    cvt = e - 3 * ivt

    partial = _k_partials(atoms_f, ids3, zeros8, pg, piv, pcv, mc, m1)
    means_f = _k_means(partial.reshape(2, MP * 8), pc)
    out_f = _k_center(atoms_f, ids3, means_f.reshape(MP, 8), ivt, cvt)
    return out_f.reshape(N, 3)


# speedup vs baseline: 16.1908x; 7.9020x over previous
import functools

import jax
import jax.numpy as jnp
from jax import lax
from jax.experimental import pallas as pl
from jax.experimental.pallas import tpu as pltpu
from jax.experimental.pallas import tpu_sc as plsc

N = 3200000
M = 100000
MP = 100352
NB = N // 128
NW = 32
CH = 64
SL = MP // 16
EM = MP * 8 // NW

_mesh = plsc.VectorSubcoreMesh(core_axis_name="c", subcore_axis_name="s")
_params = pltpu.CompilerParams(use_tc_tiling_on_sc=False, needs_layout_passes=False)
_f32 = jnp.float32
_i32 = jnp.int32


def _worker_range(w):
    lo = (w * NB) // NW
    hi = ((w + 1) * NB) // NW
    return lo, hi


@functools.partial(
    pl.kernel,
    out_type=jax.ShapeDtypeStruct((2, MP, 8), _f32),
    mesh=_mesh,
    compiler_params=_params,
    scratch_types=[
        pltpu.VMEM_SHARED((MP, 8), _f32),
        pltpu.VMEM((CH, 1, 128), _i32),
        pltpu.VMEM((CH * 128,), _f32),
        pltpu.VMEM((CH * 128,), _f32),
        pltpu.VMEM((CH * 128,), _f32),
        pltpu.VMEM((128, 8), _f32),
        pltpu.VMEM((16,), _i32),
        pltpu.VMEM((16,), _i32),
        pltpu.VMEM((16,), _f32),
        pltpu.VMEM((16,), _f32),
        pltpu.VMEM((16,), _f32),
        pltpu.VMEM((16,), _f32),
    ],
)
def _k_partials(xs, ys, zs, ids3, zeros8, piv_h, pcv_h, mx_h, my_h, mz_h, m1_h,
                pout, acc, ids_v, xs_v, ys_v, zs_v, vals_v,
                piv_v, pcv_v, mx_v, my_v, mz_v, m1_v):
    c = lax.axis_index("c")
    s = lax.axis_index("s")
    w = c * 16 + s

    zsl = pl.ds(s * SL, SL)
    pltpu.sync_copy(zeros8.at[zsl], acc.at[zsl])
    pltpu.sync_copy(piv_h, piv_v)
    pltpu.sync_copy(pcv_h, pcv_v)
    pltpu.sync_copy(mx_h, mx_v)
    pltpu.sync_copy(my_h, my_v)
    pltpu.sync_copy(mz_h, mz_v)
    pltpu.sync_copy(m1_h, m1_v)
    plsc.subcore_barrier()

    piv = piv_v[...]
    pcv = pcv_v[...]
    mx = mx_v[...]
    my = my_v[...]
    mz = mz_v[...]
    m1 = m1_v[...]

    lo, hi = _worker_range(w)
    nf = (hi - lo) // CH

    def do_block(j):
        abase = j * 128
        for t in range(64):
            idx = piv + (abase + 2 * t)
            gx = plsc.load_gather(xs_v, [idx])
            gy = plsc.load_gather(ys_v, [idx])
            gz = plsc.load_gather(zs_v, [idx])
            v = gx * mx + gy * my + gz * mz + m1
            plsc.store_scatter(vals_v, [piv + 2 * t, pcv], v)
        pltpu.sync_copy(vals_v, acc.at[ids_v.at[j, 0]], add=True)

    def chunk(k, _):
        base = lo + k * CH
        asl = pl.ds(base * 128, CH * 128)
        pltpu.sync_copy(ids3.at[pl.ds(base, CH)], ids_v)
        pltpu.sync_copy(xs.at[asl], xs_v)
        pltpu.sync_copy(ys.at[asl], ys_v)
        pltpu.sync_copy(zs.at[asl], zs_v)

        def blk(j, _):
            do_block(j)
            return 0

        lax.fori_loop(0, CH, blk, 0)
        return 0

    lax.fori_loop(0, nf, chunk, 0)

    tl = lo + nf * CH

    def tblk(j, _):
        bsl = pl.ds((tl + j) * 128, 128)
        sl0 = pl.ds(0, 128)
        pltpu.sync_copy(ids3.at[pl.ds(tl + j, 1)], ids_v.at[pl.ds(0, 1)])
        pltpu.sync_copy(xs.at[bsl], xs_v.at[sl0])
        pltpu.sync_copy(ys.at[bsl], ys_v.at[sl0])
        pltpu.sync_copy(zs.at[bsl], zs_v.at[sl0])
        do_block(0)
        return 0

    lax.fori_loop(0, hi - tl, tblk, 0)

    plsc.subcore_barrier()
    pltpu.sync_copy(acc.at[zsl], pout.at[c, zsl])


@functools.partial(
    pl.kernel,
    out_type=jax.ShapeDtypeStruct((MP * 8,), _f32),
    mesh=_mesh,
    compiler_params=_params,
    scratch_types=[
        pltpu.VMEM((EM,), _f32),
        pltpu.VMEM((EM,), _f32),
        pltpu.VMEM((16,), _i32),
    ],
)
def _k_means(p_f, pc_h, mout, s0, s1, pc_v):
    c = lax.axis_index("c")
    s = lax.axis_index("s")
    w = c * 16 + s
    off = pl.ds(w * EM, EM)
    pltpu.sync_copy(p_f.at[0, off], s0)
    pltpu.sync_copy(p_f.at[1, off], s1)
    pltpu.sync_copy(pc_h, pc_v)
    pc = pc_v[...]

    def add2(t, _):
        sl = pl.ds(t * 16, 16)
        s0[sl] = s0[sl] + s1[sl]
        return 0

    lax.fori_loop(0, EM // 16, add2, 0)

    def mean(t, _):
        sl = pl.ds(t * 16, 16)
        cnt = plsc.load_gather(s0, [pc + 16 * t])
        s0[sl] = s0[sl] / jnp.maximum(cnt, 1.0)
        return 0

    lax.fori_loop(0, EM // 16, mean, 0)
    pltpu.sync_copy(s0, mout.at[off])


@functools.partial(
    pl.kernel,
    out_type=(
        jax.ShapeDtypeStruct((N,), _f32),
        jax.ShapeDtypeStruct((N,), _f32),
        jax.ShapeDtypeStruct((N,), _f32),
    ),
    mesh=_mesh,
    compiler_params=_params,
    scratch_types=[
        pltpu.VMEM((CH, 1, 128), _i32),
        pltpu.VMEM((CH * 128,), _f32),
        pltpu.VMEM((CH * 128,), _f32),
        pltpu.VMEM((CH * 128,), _f32),
        pltpu.VMEM((CH * 128,), _f32),
        pltpu.VMEM((CH * 128,), _f32),
        pltpu.VMEM((CH * 128,), _f32),
        pltpu.VMEM((128, 8), _f32),
        pltpu.SemaphoreType.DMA,
    ],
)
def _k_center(xs, ys, zs, ids3, means8, ox, oy, oz,
              ids_v, xs_v, ys_v, zs_v, ox_v, oy_v, oz_v, mrows, sem):
    c = lax.axis_index("c")
    s = lax.axis_index("s")
    w = c * 16 + s

    lane = lax.iota(_i32, 16)
    c0 = jnp.zeros((16,), _i32)
    c1 = jnp.ones((16,), _i32)
    c2 = jnp.full((16,), 2, _i32)

    lo, hi = _worker_range(w)
    nf = (hi - lo) // CH

    def centered_block(j, _):
        pltpu.async_copy(means8.at[ids_v.at[j, 0]], mrows, sem).wait()
        abase = j * 128
        for t in range(8):
            sl = pl.ds(abase + t * 16, 16)
            iv = lane + 16 * t
            ox_v[sl] = xs_v[sl] - plsc.load_gather(mrows, [iv, c0])
            oy_v[sl] = ys_v[sl] - plsc.load_gather(mrows, [iv, c1])
            oz_v[sl] = zs_v[sl] - plsc.load_gather(mrows, [iv, c2])
        return 0

    def chunk(k, _):
        base = lo + k * CH
        asl = pl.ds(base * 128, CH * 128)
        pltpu.sync_copy(ids3.at[pl.ds(base, CH)], ids_v)
        pltpu.sync_copy(xs.at[asl], xs_v)
        pltpu.sync_copy(ys.at[asl], ys_v)
        pltpu.sync_copy(zs.at[asl], zs_v)
        lax.fori_loop(0, CH, centered_block, 0)
        pltpu.sync_copy(ox_v, ox.at[asl])
        pltpu.sync_copy(oy_v, oy.at[asl])
        pltpu.sync_copy(oz_v, oz.at[asl])
        return 0

    lax.fori_loop(0, nf, chunk, 0)

    tl = lo + nf * CH

    def tblk(j, _):
        bsl = pl.ds((tl + j) * 128, 128)
        sl0 = pl.ds(0, 128)
        pltpu.sync_copy(ids3.at[pl.ds(tl + j, 1)], ids_v.at[pl.ds(0, 1)])
        pltpu.sync_copy(xs.at[bsl], xs_v.at[sl0])
        pltpu.sync_copy(ys.at[bsl], ys_v.at[sl0])
        pltpu.sync_copy(zs.at[bsl], zs_v.at[sl0])
        lax.fori_loop(0, 1, centered_block, 0)
        pltpu.sync_copy(ox_v.at[sl0], ox.at[bsl])
        pltpu.sync_copy(oy_v.at[sl0], oy.at[bsl])
        pltpu.sync_copy(oz_v.at[sl0], oz.at[bsl])
        return 0

    lax.fori_loop(0, hi - tl, tblk, 0)


def kernel(atoms_x, graph_batch):
    xs = atoms_x[:, 0]
    ys = atoms_x[:, 1]
    zs = atoms_x[:, 2]
    ids3 = graph_batch.reshape(NB, 1, 128)
    zeros8 = jnp.zeros((MP, 8), _f32)

    lane = jnp.arange(16, dtype=_i32)
    cc = lane % 8
    piv = lane // 8
    pcv = cc
    mx = jnp.where(cc == 0, 1.0, 0.0).astype(_f32)
    my = jnp.where(cc == 1, 1.0, 0.0).astype(_f32)
    mz = jnp.where(cc == 2, 1.0, 0.0).astype(_f32)
    m1 = jnp.where(cc == 3, 1.0, 0.0).astype(_f32)
    pc = 8 * piv + 3

    partial = _k_partials(xs, ys, zs, ids3, zeros8, piv, pcv, mx, my, mz, m1)
    means_f = _k_means(partial.reshape(2, MP * 8), pc)
    ox, oy, oz = _k_center(xs, ys, zs, ids3, means_f.reshape(MP, 8))
    return jnp.stack([ox, oy, oz], axis=1)


# speedup vs baseline: 33.0697x; 2.0425x over previous
import functools

import jax
import jax.numpy as jnp
from jax import lax
from jax.experimental import pallas as pl
from jax.experimental.pallas import tpu as pltpu
from jax.experimental.pallas import tpu_sc as plsc

N = 3200000
M = 100000
MP = 100352
NB = N // 128
NW = 32
CH = 64
SL = MP // 16
EM = MP * 8 // NW

_mesh = plsc.VectorSubcoreMesh(core_axis_name="c", subcore_axis_name="s")
_params = pltpu.CompilerParams(use_tc_tiling_on_sc=False, needs_layout_passes=False)
_f32 = jnp.float32
_i32 = jnp.int32


def _worker_range(w):
    lo = (w * NB) // NW
    hi = ((w + 1) * NB) // NW
    return lo, hi


@functools.partial(
    pl.kernel,
    out_type=jax.ShapeDtypeStruct((2, MP, 8), _f32),
    mesh=_mesh,
    compiler_params=_params,
    scratch_types=[
        pltpu.VMEM_SHARED((MP, 8), _f32),
        pltpu.VMEM((CH, 1, 128), _i32),
        pltpu.VMEM((CH * 128,), _f32),
        pltpu.VMEM((CH * 128,), _f32),
        pltpu.VMEM((CH * 128,), _f32),
        pltpu.VMEM((128, 8), _f32),
        pltpu.VMEM((128, 8), _f32),
        pltpu.SemaphoreType.DMA,
        pltpu.SemaphoreType.DMA,
    ],
)
def _k_partials(xs, ys, zs, ids3, zeros8,
                pout, acc, ids_v, xs_v, ys_v, zs_v, vals_a, vals_b, sa, sb):
    c = lax.axis_index("c")
    s = lax.axis_index("s")
    w = c * 16 + s

    zsl = pl.ds(s * SL, SL)
    pltpu.sync_copy(zeros8.at[zsl], acc.at[zsl])
    plsc.subcore_barrier()

    lane = lax.iota(_i32, 16)
    c0 = jnp.zeros((16,), _i32)
    c1 = jnp.ones((16,), _i32)
    c2 = jnp.full((16,), 2, _i32)
    c3 = jnp.full((16,), 3, _i32)
    onesv = jnp.ones((16,), _f32)

    lo, hi = _worker_range(w)
    nf = (hi - lo) // CH

    def build(j, vals_v):
        abase = j * 128
        for t in range(8):
            iv = lane + 16 * t
            sl = pl.ds(abase + 16 * t, 16)
            plsc.store_scatter(vals_v, [iv, c0], xs_v[sl])
            plsc.store_scatter(vals_v, [iv, c1], ys_v[sl])
            plsc.store_scatter(vals_v, [iv, c2], zs_v[sl])
            plsc.store_scatter(vals_v, [iv, c3], onesv)

    def chunk(k, _):
        base = lo + k * CH
        asl = pl.ds(base * 128, CH * 128)
        pltpu.sync_copy(ids3.at[pl.ds(base, CH)], ids_v)
        pltpu.sync_copy(xs.at[asl], xs_v)
        pltpu.sync_copy(ys.at[asl], ys_v)
        pltpu.sync_copy(zs.at[asl], zs_v)

        def blk(p, _):
            for jo, vals_v, sem in ((0, vals_a, sa), (1, vals_b, sb)):
                j = 2 * p + jo

                @pl.when(p > 0)
                def _():
                    pltpu.make_async_copy(
                        vals_v, acc.at[ids_v.at[j, 0]], sem).wait()

                build(j, vals_v)
                pltpu.async_copy(vals_v, acc.at[ids_v.at[j, 0]], sem, add=True)
            return 0

        lax.fori_loop(0, CH // 2, blk, 0)
        pltpu.make_async_copy(vals_a, acc.at[ids_v.at[CH - 2, 0]], sa).wait()
        pltpu.make_async_copy(vals_b, acc.at[ids_v.at[CH - 1, 0]], sb).wait()
        return 0

    lax.fori_loop(0, nf, chunk, 0)

    tl = lo + nf * CH

    def tblk(j, _):
        bsl = pl.ds((tl + j) * 128, 128)
        sl0 = pl.ds(0, 128)
        pltpu.sync_copy(ids3.at[pl.ds(tl + j, 1)], ids_v.at[pl.ds(0, 1)])
        pltpu.sync_copy(xs.at[bsl], xs_v.at[sl0])
        pltpu.sync_copy(ys.at[bsl], ys_v.at[sl0])
        pltpu.sync_copy(zs.at[bsl], zs_v.at[sl0])
        build(0, vals_a)
        pltpu.sync_copy(vals_a, acc.at[ids_v.at[0, 0]], add=True)
        return 0

    lax.fori_loop(0, hi - tl, tblk, 0)

    plsc.subcore_barrier()
    pltpu.sync_copy(acc.at[zsl], pout.at[c, zsl])


@functools.partial(
    pl.kernel,
    out_type=jax.ShapeDtypeStruct((MP * 8,), _f32),
    mesh=_mesh,
    compiler_params=_params,
    scratch_types=[
        pltpu.VMEM((EM,), _f32),
        pltpu.VMEM((EM,), _f32),
        pltpu.VMEM((16,), _i32),
    ],
)
def _k_means(p_f, pc_h, mout, s0, s1, pc_v):
    c = lax.axis_index("c")
    s = lax.axis_index("s")
    w = c * 16 + s
    off = pl.ds(w * EM, EM)
    pltpu.sync_copy(p_f.at[0, off], s0)
    pltpu.sync_copy(p_f.at[1, off], s1)
    pltpu.sync_copy(pc_h, pc_v)
    pc = pc_v[...]

    def add2(t, _):
        sl = pl.ds(t * 16, 16)
        s0[sl] = s0[sl] + s1[sl]
        return 0

    lax.fori_loop(0, EM // 16, add2, 0)

    def mean(t, _):
        sl = pl.ds(t * 16, 16)
        cnt = plsc.load_gather(s0, [pc + 16 * t])
        s0[sl] = s0[sl] / jnp.maximum(cnt, 1.0)
        return 0

    lax.fori_loop(0, EM // 16, mean, 0)
    pltpu.sync_copy(s0, mout.at[off])


@functools.partial(
    pl.kernel,
    out_type=(
        jax.ShapeDtypeStruct((N,), _f32),
        jax.ShapeDtypeStruct((N,), _f32),
        jax.ShapeDtypeStruct((N,), _f32),
    ),
    mesh=_mesh,
    compiler_params=_params,
    scratch_types=[
        pltpu.VMEM((CH, 1, 128), _i32),
        pltpu.VMEM((CH * 128,), _f32),
        pltpu.VMEM((CH * 128,), _f32),
        pltpu.VMEM((CH * 128,), _f32),
        pltpu.VMEM((CH * 128,), _f32),
        pltpu.VMEM((CH * 128,), _f32),
        pltpu.VMEM((CH * 128,), _f32),
        pltpu.VMEM((128, 8), _f32),
        pltpu.VMEM((128, 8), _f32),
        pltpu.VMEM((128, 8), _f32),
        pltpu.VMEM((128, 8), _f32),
        pltpu.SemaphoreType.DMA,
        pltpu.SemaphoreType.DMA,
        pltpu.SemaphoreType.DMA,
        pltpu.SemaphoreType.DMA,
    ],
)
def _k_center(xs, ys, zs, ids3, means8, ox, oy, oz,
              ids_v, xs_v, ys_v, zs_v, ox_v, oy_v, oz_v,
              mr0, mr1, mr2, mr3, s0, s1, s2, s3):
    c = lax.axis_index("c")
    s = lax.axis_index("s")
    w = c * 16 + s

    lane = lax.iota(_i32, 16)
    c0 = jnp.zeros((16,), _i32)
    c1 = jnp.ones((16,), _i32)
    c2 = jnp.full((16,), 2, _i32)
    bufs = (mr0, mr1, mr2, mr3)
    sems = (s0, s1, s2, s3)

    lo, hi = _worker_range(w)
    nf = (hi - lo) // CH

    def compute_block(j, mrows):
        abase = j * 128
        for t in range(8):
            sl = pl.ds(abase + t * 16, 16)
            iv = lane + 16 * t
            ox_v[sl] = xs_v[sl] - plsc.load_gather(mrows, [iv, c0])
            oy_v[sl] = ys_v[sl] - plsc.load_gather(mrows, [iv, c1])
            oz_v[sl] = zs_v[sl] - plsc.load_gather(mrows, [iv, c2])

    def chunk(k, _):
        base = lo + k * CH
        asl = pl.ds(base * 128, CH * 128)
        pltpu.sync_copy(ids3.at[pl.ds(base, CH)], ids_v)
        pltpu.sync_copy(xs.at[asl], xs_v)
        pltpu.sync_copy(ys.at[asl], ys_v)
        pltpu.sync_copy(zs.at[asl], zs_v)

        for r in range(3):
            pltpu.async_copy(means8.at[ids_v.at[r, 0]], bufs[r], sems[r])

        def blk(p, _):
            for r in range(4):
                j = 4 * p + r
                pltpu.make_async_copy(
                    means8.at[ids_v.at[j, 0]], bufs[r], sems[r]).wait()

                @pl.when(j + 3 < CH)
                def _():
                    pltpu.async_copy(means8.at[ids_v.at[j + 3, 0]],
                                     bufs[(r + 3) % 4], sems[(r + 3) % 4])

                compute_block(j, bufs[r])
            return 0

        lax.fori_loop(0, CH // 4, blk, 0)
        pltpu.sync_copy(ox_v, ox.at[asl])
        pltpu.sync_copy(oy_v, oy.at[asl])
        pltpu.sync_copy(oz_v, oz.at[asl])
        return 0

    lax.fori_loop(0, nf, chunk, 0)

    tl = lo + nf * CH

    def tblk(j, _):
        bsl = pl.ds((tl + j) * 128, 128)
        sl0 = pl.ds(0, 128)
        pltpu.sync_copy(ids3.at[pl.ds(tl + j, 1)], ids_v.at[pl.ds(0, 1)])
        pltpu.sync_copy(xs.at[bsl], xs_v.at[sl0])
        pltpu.sync_copy(ys.at[bsl], ys_v.at[sl0])
        pltpu.sync_copy(zs.at[bsl], zs_v.at[sl0])
        pltpu.async_copy(means8.at[ids_v.at[0, 0]], mr0, s0).wait()
        compute_block(0, mr0)
        pltpu.sync_copy(ox_v.at[sl0], ox.at[bsl])
        pltpu.sync_copy(oy_v.at[sl0], oy.at[bsl])
        pltpu.sync_copy(oz_v.at[sl0], oz.at[bsl])
        return 0

    lax.fori_loop(0, hi - tl, tblk, 0)


def kernel(atoms_x, graph_batch):
    xs = atoms_x[:, 0]
    ys = atoms_x[:, 1]
    zs = atoms_x[:, 2]
    ids3 = graph_batch.reshape(NB, 1, 128)
    zeros8 = jnp.zeros((MP, 8), _f32)

    lane = jnp.arange(16, dtype=_i32)
    pc = 8 * (lane // 8) + 3

    partial = _k_partials(xs, ys, zs, ids3, zeros8)
    means_f = _k_means(partial.reshape(2, MP * 8), pc)
    ox, oy, oz = _k_center(xs, ys, zs, ids3, means_f.reshape(MP, 8))
    return jnp.stack([ox, oy, oz], axis=1)


# speedup vs baseline: 51.3173x; 1.5518x over previous
import functools

import jax
import jax.numpy as jnp
from jax import lax
from jax.experimental import pallas as pl
from jax.experimental.pallas import tpu as pltpu
from jax.experimental.pallas import tpu_sc as plsc

N = 3200000
M = 100000
MP = 100352
NB = N // 128
NW = 32
CH = 64
SL = MP // 16
EM = MP * 8 // NW

_mesh = plsc.VectorSubcoreMesh(core_axis_name="c", subcore_axis_name="s")
_params = pltpu.CompilerParams(use_tc_tiling_on_sc=False, needs_layout_passes=False)
_f32 = jnp.float32
_i32 = jnp.int32


def _worker_range(w):
    lo = (w * NB) // NW
    hi = ((w + 1) * NB) // NW
    return lo, hi


@functools.partial(
    pl.kernel,
    out_type=jax.ShapeDtypeStruct((2, MP, 8), _f32),
    mesh=_mesh,
    compiler_params=_params,
    scratch_types=[
        pltpu.VMEM_SHARED((MP, 8), _f32),
        pltpu.VMEM((CH, 1, 128), _i32),
        pltpu.VMEM((CH * 128,), _f32),
        pltpu.VMEM((CH * 128,), _f32),
        pltpu.VMEM((CH * 128,), _f32),
        pltpu.VMEM((128, 8), _f32),
        pltpu.VMEM((128, 8), _f32),
        pltpu.SemaphoreType.DMA,
        pltpu.SemaphoreType.DMA,
    ],
)
def _k_partials(xs, ys, zs, ids3, zeros8,
                pout, acc, ids_v, xs_v, ys_v, zs_v, vals_a, vals_b, sa, sb):
    c = lax.axis_index("c")
    s = lax.axis_index("s")
    w = c * 16 + s

    zsl = pl.ds(s * SL, SL)
    pltpu.sync_copy(zeros8.at[zsl], acc.at[zsl])
    plsc.subcore_barrier()

    lane = lax.iota(_i32, 16)
    c0 = jnp.zeros((16,), _i32)
    c1 = jnp.ones((16,), _i32)
    c2 = jnp.full((16,), 2, _i32)
    c3 = jnp.full((16,), 3, _i32)
    onesv = jnp.ones((16,), _f32)

    lo, hi = _worker_range(w)
    nf = (hi - lo) // CH

    def build(j, vals_v):
        abase = j * 128
        for t in range(8):
            iv = lane + 16 * t
            sl = pl.ds(abase + 16 * t, 16)
            plsc.store_scatter(vals_v, [iv, c0], xs_v[sl])
            plsc.store_scatter(vals_v, [iv, c1], ys_v[sl])
            plsc.store_scatter(vals_v, [iv, c2], zs_v[sl])
            plsc.store_scatter(vals_v, [iv, c3], onesv)

    def chunk(k, _):
        base = lo + k * CH
        asl = pl.ds(base * 128, CH * 128)
        pltpu.sync_copy(ids3.at[pl.ds(base, CH)], ids_v)
        pltpu.sync_copy(xs.at[asl], xs_v)
        pltpu.sync_copy(ys.at[asl], ys_v)
        pltpu.sync_copy(zs.at[asl], zs_v)

        def blk(p, _):
            for jo, vals_v, sem in ((0, vals_a, sa), (1, vals_b, sb)):
                j = 2 * p + jo

                @pl.when(p > 0)
                def _():
                    pltpu.make_async_copy(
                        vals_v, acc.at[ids_v.at[j, 0]], sem).wait()

                build(j, vals_v)
                pltpu.async_copy(vals_v, acc.at[ids_v.at[j, 0]], sem, add=True)
            return 0

        lax.fori_loop(0, CH // 2, blk, 0)
        pltpu.make_async_copy(vals_a, acc.at[ids_v.at[CH - 2, 0]], sa).wait()
        pltpu.make_async_copy(vals_b, acc.at[ids_v.at[CH - 1, 0]], sb).wait()
        return 0

    lax.fori_loop(0, nf, chunk, 0)

    tl = lo + nf * CH

    def tblk(j, _):
        bsl = pl.ds((tl + j) * 128, 128)
        sl0 = pl.ds(0, 128)
        pltpu.sync_copy(ids3.at[pl.ds(tl + j, 1)], ids_v.at[pl.ds(0, 1)])
        pltpu.sync_copy(xs.at[bsl], xs_v.at[sl0])
        pltpu.sync_copy(ys.at[bsl], ys_v.at[sl0])
        pltpu.sync_copy(zs.at[bsl], zs_v.at[sl0])
        build(0, vals_a)
        pltpu.sync_copy(vals_a, acc.at[ids_v.at[0, 0]], add=True)
        return 0

    lax.fori_loop(0, hi - tl, tblk, 0)

    plsc.subcore_barrier()
    pltpu.sync_copy(acc.at[zsl], pout.at[c, zsl])


W = 2048


@functools.partial(
    pl.kernel,
    out_type=(
        jax.ShapeDtypeStruct((N,), _f32),
        jax.ShapeDtypeStruct((N,), _f32),
        jax.ShapeDtypeStruct((N,), _f32),
    ),
    mesh=_mesh,
    compiler_params=_params,
    scratch_types=[
        pltpu.VMEM((CH, 1, 128), _i32),
        pltpu.VMEM((CH * 128,), _f32),
        pltpu.VMEM((CH * 128,), _f32),
        pltpu.VMEM((CH * 128,), _f32),
        pltpu.VMEM((CH * 128,), _f32),
        pltpu.VMEM((CH * 128,), _f32),
        pltpu.VMEM((CH * 128,), _f32),
        pltpu.VMEM((W, 8), _f32),
        pltpu.VMEM((W, 8), _f32),
        pltpu.VMEM((128, 8), _f32),
        pltpu.VMEM((128, 8), _f32),
        pltpu.SemaphoreType.DMA,
        pltpu.SemaphoreType.DMA,
    ],
)
def _k_center(xs, ys, zs, ids3, part8, ox, oy, oz,
              ids_v, xs_v, ys_v, zs_v, ox_v, oy_v, oz_v,
              pa, pb, mra, mrb, sa, sb):
    c = lax.axis_index("c")
    s = lax.axis_index("s")
    w = c * 16 + s

    lane = lax.iota(_i32, 16)
    c0 = jnp.zeros((16,), _i32)
    c1 = jnp.ones((16,), _i32)
    c2 = jnp.full((16,), 2, _i32)
    c3 = jnp.full((16,), 3, _i32)
    pr2 = lane // 8
    pcol = lane - 8 * pr2

    lo, hi = _worker_range(w)
    nc = (hi - lo + CH - 1) // CH

    def chunk(k, _):
        base = jnp.minimum(lo + k * CH, hi - CH)
        asl = pl.ds(base * 128, CH * 128)
        pltpu.sync_copy(ids3.at[pl.ds(base, CH)], ids_v)
        pltpu.sync_copy(xs.at[asl], xs_v)
        pltpu.sync_copy(ys.at[asl], ys_v)
        pltpu.sync_copy(zs.at[asl], zs_v)

        start = jnp.minimum(ids_v[0, 0, pl.ds(0, 16)][0], MP - W)
        need = ids_v[CH - 1, 0, pl.ds(112, 16)][15] - start + 1

        @pl.when(need <= W)
        def _fast():
            psl = pl.ds(start, W)
            pltpu.sync_copy(part8.at[0, psl, :], pa)
            pltpu.sync_copy(part8.at[1, psl, :], pb)

            def comb(t, _):
                rows = pr2 + 2 * t
                v = (plsc.load_gather(pa, [rows, pcol])
                     + plsc.load_gather(pb, [rows, pcol]))
                plsc.store_scatter(pa, [rows, pcol], v)
                cnt = plsc.load_gather(pa, [rows, c3])
                plsc.store_scatter(pa, [rows, pcol], v / jnp.maximum(cnt, 1.0))
                return 0

            lax.fori_loop(0, (need + 1) // 2, comb, 0)

            def blk(j, _):
                abase = j * 128
                for t in range(8):
                    sl = pl.ds(abase + t * 16, 16)
                    rel = ids_v[j, 0, pl.ds(16 * t, 16)] - start
                    ox_v[sl] = xs_v[sl] - plsc.load_gather(pa, [rel, c0])
                    oy_v[sl] = ys_v[sl] - plsc.load_gather(pa, [rel, c1])
                    oz_v[sl] = zs_v[sl] - plsc.load_gather(pa, [rel, c2])
                return 0

            lax.fori_loop(0, CH, blk, 0)

        @pl.when(need > W)
        def _slow():
            def blk(j, _):
                da = pltpu.async_copy(part8.at[0].at[ids_v.at[j, 0]], mra, sa)
                db = pltpu.async_copy(part8.at[1].at[ids_v.at[j, 0]], mrb, sb)
                da.wait()
                db.wait()
                abase = j * 128
                for t in range(8):
                    sl = pl.ds(abase + t * 16, 16)
                    iv = lane + 16 * t
                    cnt = (plsc.load_gather(mra, [iv, c3])
                           + plsc.load_gather(mrb, [iv, c3]))
                    inv = 1.0 / jnp.maximum(cnt, 1.0)
                    mxv = (plsc.load_gather(mra, [iv, c0])
                           + plsc.load_gather(mrb, [iv, c0])) * inv
                    myv = (plsc.load_gather(mra, [iv, c1])
                           + plsc.load_gather(mrb, [iv, c1])) * inv
                    mzv = (plsc.load_gather(mra, [iv, c2])
                           + plsc.load_gather(mrb, [iv, c2])) * inv
                    ox_v[sl] = xs_v[sl] - mxv
                    oy_v[sl] = ys_v[sl] - myv
                    oz_v[sl] = zs_v[sl] - mzv
                return 0

            lax.fori_loop(0, CH, blk, 0)

        pltpu.sync_copy(ox_v, ox.at[asl])
        pltpu.sync_copy(oy_v, oy.at[asl])
        pltpu.sync_copy(oz_v, oz.at[asl])
        return 0

    lax.fori_loop(0, nc, chunk, 0)


def kernel(atoms_x, graph_batch):
    xs = atoms_x[:, 0]
    ys = atoms_x[:, 1]
    zs = atoms_x[:, 2]
    ids3 = graph_batch.reshape(NB, 1, 128)
    zeros8 = jnp.zeros((MP, 8), _f32)

    partial = _k_partials(xs, ys, zs, ids3, zeros8)
    ox, oy, oz = _k_center(xs, ys, zs, ids3, partial)
    return jnp.stack([ox, oy, oz], axis=1)


# speedup vs baseline: 51.7016x; 1.0075x over previous
import functools

import jax
import jax.numpy as jnp
from jax import lax
from jax.experimental import pallas as pl
from jax.experimental.pallas import tpu as pltpu
from jax.experimental.pallas import tpu_sc as plsc

N = 3200000
M = 100000
MP = 100352
NB = N // 128
NW = 32
CH = 64
SL = MP // 16
EM = MP * 8 // NW

_mesh = plsc.VectorSubcoreMesh(core_axis_name="c", subcore_axis_name="s")
_params = pltpu.CompilerParams(use_tc_tiling_on_sc=False, needs_layout_passes=False)
_f32 = jnp.float32
_i32 = jnp.int32


def _worker_range(w):
    lo = (w * NB) // NW
    hi = ((w + 1) * NB) // NW
    return lo, hi


@functools.partial(
    pl.kernel,
    out_type=jax.ShapeDtypeStruct((2, MP, 8), _f32),
    mesh=_mesh,
    compiler_params=_params,
    scratch_types=[
        pltpu.VMEM_SHARED((MP, 8), _f32),
        pltpu.VMEM((CH, 1, 128), _i32),
        pltpu.VMEM((CH * 128,), _f32),
        pltpu.VMEM((CH * 128,), _f32),
        pltpu.VMEM((CH * 128,), _f32),
        pltpu.VMEM((128, 8), _f32),
        pltpu.VMEM((128, 8), _f32),
        pltpu.VMEM((128, 8), _f32),
        pltpu.VMEM((128, 8), _f32),
        pltpu.SemaphoreType.DMA,
        pltpu.SemaphoreType.DMA,
        pltpu.SemaphoreType.DMA,
        pltpu.SemaphoreType.DMA,
    ],
)
def _k_partials(xs, ys, zs, ids3, zeros8,
                pout, acc, ids_v, xs_v, ys_v, zs_v,
                vals_a, vals_b, vals_c, vals_d, sa, sb, sc_, sd):
    c = lax.axis_index("c")
    s = lax.axis_index("s")
    w = c * 16 + s

    zsl = pl.ds(s * SL, SL)
    pltpu.sync_copy(zeros8.at[zsl], acc.at[zsl])
    plsc.subcore_barrier()

    lane = lax.iota(_i32, 16)
    c0 = jnp.zeros((16,), _i32)
    c1 = jnp.ones((16,), _i32)
    c2 = jnp.full((16,), 2, _i32)
    c3 = jnp.full((16,), 3, _i32)
    onesv = jnp.ones((16,), _f32)

    lo, hi = _worker_range(w)
    nf = (hi - lo) // CH

    def build(j, vals_v):
        abase = j * 128
        for t in range(8):
            iv = lane + 16 * t
            sl = pl.ds(abase + 16 * t, 16)
            plsc.store_scatter(vals_v, [iv, c0], xs_v[sl])
            plsc.store_scatter(vals_v, [iv, c1], ys_v[sl])
            plsc.store_scatter(vals_v, [iv, c2], zs_v[sl])
            plsc.store_scatter(vals_v, [iv, c3], onesv)

    def chunk(k, _):
        base = lo + k * CH
        asl = pl.ds(base * 128, CH * 128)
        pltpu.sync_copy(ids3.at[pl.ds(base, CH)], ids_v)
        pltpu.sync_copy(xs.at[asl], xs_v)
        pltpu.sync_copy(ys.at[asl], ys_v)
        pltpu.sync_copy(zs.at[asl], zs_v)

        def blk(p, _):
            for jo, vals_v, sem in ((0, vals_a, sa), (1, vals_b, sb),
                                    (2, vals_c, sc_), (3, vals_d, sd)):
                j = 4 * p + jo

                @pl.when(p > 0)
                def _():
                    pltpu.make_async_copy(
                        vals_v, acc.at[ids_v.at[j, 0]], sem).wait()

                build(j, vals_v)
                pltpu.async_copy(vals_v, acc.at[ids_v.at[j, 0]], sem, add=True)
            return 0

        lax.fori_loop(0, CH // 4, blk, 0)
        pltpu.make_async_copy(vals_a, acc.at[ids_v.at[CH - 4, 0]], sa).wait()
        pltpu.make_async_copy(vals_b, acc.at[ids_v.at[CH - 3, 0]], sb).wait()
        pltpu.make_async_copy(vals_c, acc.at[ids_v.at[CH - 2, 0]], sc_).wait()
        pltpu.make_async_copy(vals_d, acc.at[ids_v.at[CH - 1, 0]], sd).wait()
        return 0

    lax.fori_loop(0, nf, chunk, 0)

    tl = lo + nf * CH

    def tblk(j, _):
        bsl = pl.ds((tl + j) * 128, 128)
        sl0 = pl.ds(0, 128)
        pltpu.sync_copy(ids3.at[pl.ds(tl + j, 1)], ids_v.at[pl.ds(0, 1)])
        pltpu.sync_copy(xs.at[bsl], xs_v.at[sl0])
        pltpu.sync_copy(ys.at[bsl], ys_v.at[sl0])
        pltpu.sync_copy(zs.at[bsl], zs_v.at[sl0])
        build(0, vals_a)
        pltpu.sync_copy(vals_a, acc.at[ids_v.at[0, 0]], add=True)
        return 0

    lax.fori_loop(0, hi - tl, tblk, 0)

    plsc.subcore_barrier()
    pltpu.sync_copy(acc.at[zsl], pout.at[c, zsl])


W = 2048


@functools.partial(
    pl.kernel,
    out_type=(
        jax.ShapeDtypeStruct((N,), _f32),
        jax.ShapeDtypeStruct((N,), _f32),
        jax.ShapeDtypeStruct((N,), _f32),
    ),
    mesh=_mesh,
    compiler_params=_params,
    scratch_types=[
        pltpu.VMEM((CH, 1, 128), _i32),
        pltpu.VMEM((CH * 128,), _f32),
        pltpu.VMEM((CH * 128,), _f32),
        pltpu.VMEM((CH * 128,), _f32),
        pltpu.VMEM((CH * 128,), _f32),
        pltpu.VMEM((CH * 128,), _f32),
        pltpu.VMEM((CH * 128,), _f32),
        pltpu.VMEM((W, 8), _f32),
        pltpu.VMEM((W, 8), _f32),
        pltpu.VMEM((128, 8), _f32),
        pltpu.VMEM((128, 8), _f32),
        pltpu.SemaphoreType.DMA,
        pltpu.SemaphoreType.DMA,
    ],
)
def _k_center(xs, ys, zs, ids3, part8, ox, oy, oz,
              ids_v, xs_v, ys_v, zs_v, ox_v, oy_v, oz_v,
              pa, pb, mra, mrb, sa, sb):
    c = lax.axis_index("c")
    s = lax.axis_index("s")
    w = c * 16 + s

    lane = lax.iota(_i32, 16)
    c0 = jnp.zeros((16,), _i32)
    c1 = jnp.ones((16,), _i32)
    c2 = jnp.full((16,), 2, _i32)
    c3 = jnp.full((16,), 3, _i32)
    pr2 = lane // 8
    pcol = lane - 8 * pr2

    lo, hi = _worker_range(w)
    nc = (hi - lo + CH - 1) // CH

    def chunk(k, _):
        base = jnp.minimum(lo + k * CH, hi - CH)
        asl = pl.ds(base * 128, CH * 128)
        pltpu.sync_copy(ids3.at[pl.ds(base, CH)], ids_v)
        pltpu.sync_copy(xs.at[asl], xs_v)
        pltpu.sync_copy(ys.at[asl], ys_v)
        pltpu.sync_copy(zs.at[asl], zs_v)

        start = jnp.minimum(ids_v[0, 0, pl.ds(0, 16)][0], MP - W)
        need = ids_v[CH - 1, 0, pl.ds(112, 16)][15] - start + 1

        @pl.when(need <= W)
        def _fast():
            psl = pl.ds(start, W)
            pltpu.sync_copy(part8.at[0, psl, :], pa)
            pltpu.sync_copy(part8.at[1, psl, :], pb)

            def comb(t, _):
                rows = pr2 + 2 * t
                v = (plsc.load_gather(pa, [rows, pcol])
                     + plsc.load_gather(pb, [rows, pcol]))
                plsc.store_scatter(pa, [rows, pcol], v)
                cnt = plsc.load_gather(pa, [rows, c3])
                plsc.store_scatter(pa, [rows, pcol], v / jnp.maximum(cnt, 1.0))
                return 0

            lax.fori_loop(0, (need + 1) // 2, comb, 0)

            def blk(j, _):
                abase = j * 128
                for t in range(8):
                    sl = pl.ds(abase + t * 16, 16)
                    rel = ids_v[j, 0, pl.ds(16 * t, 16)] - start
                    ox_v[sl] = xs_v[sl] - plsc.load_gather(pa, [rel, c0])
                    oy_v[sl] = ys_v[sl] - plsc.load_gather(pa, [rel, c1])
                    oz_v[sl] = zs_v[sl] - plsc.load_gather(pa, [rel, c2])
                return 0

            lax.fori_loop(0, CH, blk, 0)

        @pl.when(need > W)
        def _slow():
            def blk(j, _):
                da = pltpu.async_copy(part8.at[0].at[ids_v.at[j, 0]], mra, sa)
                db = pltpu.async_copy(part8.at[1].at[ids_v.at[j, 0]], mrb, sb)
                da.wait()
                db.wait()
                abase = j * 128
                for t in range(8):
                    sl = pl.ds(abase + t * 16, 16)
                    iv = lane + 16 * t
                    cnt = (plsc.load_gather(mra, [iv, c3])
                           + plsc.load_gather(mrb, [iv, c3]))
                    inv = 1.0 / jnp.maximum(cnt, 1.0)
                    mxv = (plsc.load_gather(mra, [iv, c0])
                           + plsc.load_gather(mrb, [iv, c0])) * inv
                    myv = (plsc.load_gather(mra, [iv, c1])
                           + plsc.load_gather(mrb, [iv, c1])) * inv
                    mzv = (plsc.load_gather(mra, [iv, c2])
                           + plsc.load_gather(mrb, [iv, c2])) * inv
                    ox_v[sl] = xs_v[sl] - mxv
                    oy_v[sl] = ys_v[sl] - myv
                    oz_v[sl] = zs_v[sl] - mzv
                return 0

            lax.fori_loop(0, CH, blk, 0)

        pltpu.sync_copy(ox_v, ox.at[asl])
        pltpu.sync_copy(oy_v, oy.at[asl])
        pltpu.sync_copy(oz_v, oz.at[asl])
        return 0

    lax.fori_loop(0, nc, chunk, 0)


def kernel(atoms_x, graph_batch):
    xs = atoms_x[:, 0]
    ys = atoms_x[:, 1]
    zs = atoms_x[:, 2]
    ids3 = graph_batch.reshape(NB, 1, 128)
    zeros8 = jnp.zeros((MP, 8), _f32)

    partial = _k_partials(xs, ys, zs, ids3, zeros8)
    ox, oy, oz = _k_center(xs, ys, zs, ids3, partial)
    return jnp.stack([ox, oy, oz], axis=1)


# speedup vs baseline: 52.0451x; 1.0066x over previous
import functools

import jax
import jax.numpy as jnp
from jax import lax
from jax.experimental import pallas as pl
from jax.experimental.pallas import tpu as pltpu
from jax.experimental.pallas import tpu_sc as plsc

N = 3200000
M = 100000
MP = 100352
NB = N // 128
NW = 32
CH = 64
SL = MP // 16
EM = MP * 8 // NW

_mesh = plsc.VectorSubcoreMesh(core_axis_name="c", subcore_axis_name="s")
_params = pltpu.CompilerParams(use_tc_tiling_on_sc=False, needs_layout_passes=False)
_f32 = jnp.float32
_i32 = jnp.int32


def _worker_range(w):
    lo = (w * NB) // NW
    hi = ((w + 1) * NB) // NW
    return lo, hi


@functools.partial(
    pl.kernel,
    out_type=jax.ShapeDtypeStruct((2, MP, 8), _f32),
    mesh=_mesh,
    compiler_params=_params,
    scratch_types=[
        pltpu.VMEM_SHARED((MP, 8), _f32),
        pltpu.VMEM((CH, 1, 128), _i32),
        pltpu.VMEM((CH * 128,), _f32),
        pltpu.VMEM((CH * 128,), _f32),
        pltpu.VMEM((CH * 128,), _f32),
        pltpu.VMEM((128, 8), _f32),
        pltpu.VMEM((128, 8), _f32),
        pltpu.VMEM((128, 8), _f32),
        pltpu.VMEM((128, 8), _f32),
        pltpu.SemaphoreType.DMA,
        pltpu.SemaphoreType.DMA,
        pltpu.SemaphoreType.DMA,
        pltpu.SemaphoreType.DMA,
    ],
)
def _k_partials(xs, ys, zs, ids3, zeros8,
                pout, acc, ids_v, xs_v, ys_v, zs_v,
                vals_a, vals_b, vals_c, vals_d, sa, sb, sc_, sd):
    c = lax.axis_index("c")
    s = lax.axis_index("s")
    w = c * 16 + s

    zsl = pl.ds(s * SL, SL)
    pltpu.sync_copy(zeros8.at[zsl], acc.at[zsl])
    plsc.subcore_barrier()

    lane = lax.iota(_i32, 16)
    c0 = jnp.zeros((16,), _i32)
    c1 = jnp.ones((16,), _i32)
    c2 = jnp.full((16,), 2, _i32)
    c3 = jnp.full((16,), 3, _i32)
    onesv = jnp.ones((16,), _f32)

    lo, hi = _worker_range(w)
    nf = (hi - lo) // CH

    for vals_v in (vals_a, vals_b, vals_c, vals_d):
        for t in range(8):
            plsc.store_scatter(vals_v, [lane + 16 * t, c3], onesv)

    def build(j, vals_v):
        abase = j * 128
        for t in range(8):
            iv = lane + 16 * t
            sl = pl.ds(abase + 16 * t, 16)
            plsc.store_scatter(vals_v, [iv, c0], xs_v[sl])
            plsc.store_scatter(vals_v, [iv, c1], ys_v[sl])
            plsc.store_scatter(vals_v, [iv, c2], zs_v[sl])

    def chunk(k, _):
        base = lo + k * CH
        asl = pl.ds(base * 128, CH * 128)
        pltpu.sync_copy(ids3.at[pl.ds(base, CH)], ids_v)
        pltpu.sync_copy(xs.at[asl], xs_v)
        pltpu.sync_copy(ys.at[asl], ys_v)
        pltpu.sync_copy(zs.at[asl], zs_v)

        def blk(p, _):
            for jo, vals_v, sem in ((0, vals_a, sa), (1, vals_b, sb),
                                    (2, vals_c, sc_), (3, vals_d, sd)):
                j = 4 * p + jo

                @pl.when(p > 0)
                def _():
                    pltpu.make_async_copy(
                        vals_v, acc.at[ids_v.at[j, 0]], sem).wait()

                build(j, vals_v)
                pltpu.async_copy(vals_v, acc.at[ids_v.at[j, 0]], sem, add=True)
            return 0

        lax.fori_loop(0, CH // 4, blk, 0)
        pltpu.make_async_copy(vals_a, acc.at[ids_v.at[CH - 4, 0]], sa).wait()
        pltpu.make_async_copy(vals_b, acc.at[ids_v.at[CH - 3, 0]], sb).wait()
        pltpu.make_async_copy(vals_c, acc.at[ids_v.at[CH - 2, 0]], sc_).wait()
        pltpu.make_async_copy(vals_d, acc.at[ids_v.at[CH - 1, 0]], sd).wait()
        return 0

    lax.fori_loop(0, nf, chunk, 0)

    tl = lo + nf * CH

    def tblk(j, _):
        bsl = pl.ds((tl + j) * 128, 128)
        sl0 = pl.ds(0, 128)
        pltpu.sync_copy(ids3.at[pl.ds(tl + j, 1)], ids_v.at[pl.ds(0, 1)])
        pltpu.sync_copy(xs.at[bsl], xs_v.at[sl0])
        pltpu.sync_copy(ys.at[bsl], ys_v.at[sl0])
        pltpu.sync_copy(zs.at[bsl], zs_v.at[sl0])
        build(0, vals_a)
        pltpu.sync_copy(vals_a, acc.at[ids_v.at[0, 0]], add=True)
        return 0

    lax.fori_loop(0, hi - tl, tblk, 0)

    plsc.subcore_barrier()
    pltpu.sync_copy(acc.at[zsl], pout.at[c, zsl])


W = 2048


@functools.partial(
    pl.kernel,
    out_type=(
        jax.ShapeDtypeStruct((N,), _f32),
        jax.ShapeDtypeStruct((N,), _f32),
        jax.ShapeDtypeStruct((N,), _f32),
    ),
    mesh=_mesh,
    compiler_params=_params,
    scratch_types=[
        pltpu.VMEM((CH, 1, 128), _i32),
        pltpu.VMEM((CH * 128,), _f32),
        pltpu.VMEM((CH * 128,), _f32),
        pltpu.VMEM((CH * 128,), _f32),
        pltpu.VMEM((CH * 128,), _f32),
        pltpu.VMEM((CH * 128,), _f32),
        pltpu.VMEM((CH * 128,), _f32),
        pltpu.VMEM((W, 8), _f32),
        pltpu.VMEM((W, 8), _f32),
        pltpu.VMEM((128, 8), _f32),
        pltpu.VMEM((128, 8), _f32),
        pltpu.SemaphoreType.DMA,
        pltpu.SemaphoreType.DMA,
    ],
)
def _k_center(xs, ys, zs, ids3, part8, ox, oy, oz,
              ids_v, xs_v, ys_v, zs_v, ox_v, oy_v, oz_v,
              pa, pb, mra, mrb, sa, sb):
    c = lax.axis_index("c")
    s = lax.axis_index("s")
    w = c * 16 + s

    lane = lax.iota(_i32, 16)
    c0 = jnp.zeros((16,), _i32)
    c1 = jnp.ones((16,), _i32)
    c2 = jnp.full((16,), 2, _i32)
    c3 = jnp.full((16,), 3, _i32)
    pr2 = lane // 8
    pcol = lane - 8 * pr2

    lo, hi = _worker_range(w)
    nc = (hi - lo + CH - 1) // CH

    def chunk(k, _):
        base = jnp.minimum(lo + k * CH, hi - CH)
        asl = pl.ds(base * 128, CH * 128)
        pltpu.sync_copy(ids3.at[pl.ds(base, CH)], ids_v)
        pltpu.sync_copy(xs.at[asl], xs_v)
        pltpu.sync_copy(ys.at[asl], ys_v)
        pltpu.sync_copy(zs.at[asl], zs_v)

        start = jnp.minimum(ids_v[0, 0, pl.ds(0, 16)][0], MP - W)
        need = ids_v[CH - 1, 0, pl.ds(112, 16)][15] - start + 1

        @pl.when(need <= W)
        def _fast():
            psl = pl.ds(start, W)
            pltpu.sync_copy(part8.at[0, psl, :], pa)
            pltpu.sync_copy(part8.at[1, psl, :], pb)

            def comb(t, _):
                rows = pr2 + 2 * t
                v = (plsc.load_gather(pa, [rows, pcol])
                     + plsc.load_gather(pb, [rows, pcol]))
                plsc.store_scatter(pa, [rows, pcol], v)
                cnt = plsc.load_gather(pa, [rows, c3])
                plsc.store_scatter(pa, [rows, pcol], v / jnp.maximum(cnt, 1.0))
                return 0

            lax.fori_loop(0, (need + 1) // 2, comb, 0)

            def blk(j, _):
                abase = j * 128
                for t in range(8):
                    sl = pl.ds(abase + t * 16, 16)
                    rel = ids_v[j, 0, pl.ds(16 * t, 16)] - start
                    ox_v[sl] = xs_v[sl] - plsc.load_gather(pa, [rel, c0])
                    oy_v[sl] = ys_v[sl] - plsc.load_gather(pa, [rel, c1])
                    oz_v[sl] = zs_v[sl] - plsc.load_gather(pa, [rel, c2])
                return 0

            lax.fori_loop(0, CH, blk, 0)

        @pl.when(need > W)
        def _slow():
            def blk(j, _):
                da = pltpu.async_copy(part8.at[0].at[ids_v.at[j, 0]], mra, sa)
                db = pltpu.async_copy(part8.at[1].at[ids_v.at[j, 0]], mrb, sb)
                da.wait()
                db.wait()
                abase = j * 128
                for t in range(8):
                    sl = pl.ds(abase + t * 16, 16)
                    iv = lane + 16 * t
                    cnt = (plsc.load_gather(mra, [iv, c3])
                           + plsc.load_gather(mrb, [iv, c3]))
                    inv = 1.0 / jnp.maximum(cnt, 1.0)
                    mxv = (plsc.load_gather(mra, [iv, c0])
                           + plsc.load_gather(mrb, [iv, c0])) * inv
                    myv = (plsc.load_gather(mra, [iv, c1])
                           + plsc.load_gather(mrb, [iv, c1])) * inv
                    mzv = (plsc.load_gather(mra, [iv, c2])
                           + plsc.load_gather(mrb, [iv, c2])) * inv
                    ox_v[sl] = xs_v[sl] - mxv
                    oy_v[sl] = ys_v[sl] - myv
                    oz_v[sl] = zs_v[sl] - mzv
                return 0

            lax.fori_loop(0, CH, blk, 0)

        pltpu.sync_copy(ox_v, ox.at[asl])
        pltpu.sync_copy(oy_v, oy.at[asl])
        pltpu.sync_copy(oz_v, oz.at[asl])
        return 0

    lax.fori_loop(0, nc, chunk, 0)


def kernel(atoms_x, graph_batch):
    xs = atoms_x[:, 0]
    ys = atoms_x[:, 1]
    zs = atoms_x[:, 2]
    ids3 = graph_batch.reshape(NB, 1, 128)
    zeros8 = jnp.zeros((MP, 8), _f32)

    partial = _k_partials(xs, ys, zs, ids3, zeros8)
    ox, oy, oz = _k_center(xs, ys, zs, ids3, partial)
    return jnp.stack([ox, oy, oz], axis=1)
